# baseline (speedup 1.0000x reference)
import functools

import jax
import jax.numpy as jnp
import numpy as np
from jax import lax
from jax.experimental import pallas as pl
from jax.experimental.pallas import tpu as pltpu
from jax.experimental.pallas import tpu_sc as plsc

ROWS = 64
VOCAB = 100000
QBASE = 24960
QW = 25088
CW = 1792
NCHUNK = QW // CW
TAIL0 = 99968
TAILW = VOCAB - TAIL0
STREAMS = 4
GROUPS = CW // 16 // STREAMS
NWORKERS = 32
GROWS = 8


def _threefry2x32(k0, k1, x0, x1):
    def rotl(x, r):
        return ((x << np.uint32(r)) | (x >> np.uint32(32 - r))).astype(np.uint32)
    rot = ((13, 15, 26, 6), (17, 29, 16, 24))
    ks = [np.uint32(k0), np.uint32(k1),
          np.uint32(np.uint32(k0) ^ np.uint32(k1) ^ np.uint32(0x1BD11BDA))]
    x0 = (x0 + ks[0]).astype(np.uint32)
    x1 = (x1 + ks[1]).astype(np.uint32)
    for i in range(5):
        for r in rot[i % 2]:
            x0 = (x0 + x1).astype(np.uint32)
            x1 = rotl(x1, r)
            x1 = x1 ^ x0
        x0 = (x0 + ks[(i + 1) % 3]).astype(np.uint32)
        x1 = (x1 + ks[(i + 2) % 3] + np.uint32(i + 1)).astype(np.uint32)
    return x0, x1


def _compute_ln_noise() -> np.ndarray:
    size = ROWS * VOCAB
    idx = np.arange(size, dtype=np.uint64)
    x0 = (idx >> np.uint64(32)).astype(np.uint32)
    x1 = (idx & np.uint64(0xFFFFFFFF)).astype(np.uint32)
    r0, r1 = _threefry2x32(np.uint32(0), np.uint32(42), x0, x1)
    bits = r0 ^ r1
    u = ((bits >> np.uint32(9)) | np.uint32(0x3F800000)).view(np.float32) \
        - np.float32(1.0)
    u = np.maximum(u, np.float32(0.0))
    noise = (-np.log1p(-u)).astype(np.float32)
    noise = np.maximum(noise, np.float32(1e-10))
    ln = np.log(noise).astype(np.float32)
    return np.concatenate([ln, np.zeros(VOCAB, np.float32)])


_LN_NOISE = _compute_ln_noise()


def _merge(vm, im, vm2, im2):
    take = (vm2 > vm) | ((vm2 == vm) & (im2 < im))
    return jnp.where(take, vm2, vm), jnp.where(take, im2, im)


def _sampler_body(logits_hbm, ln_hbm, tail_hbm, a_hbm, rs_hbm,
                  outm_hbm, outt_hbm,
                  lbuf0, lbuf1, nbuf0, nbuf1, abuf, rsbuf,
                  tlbuf, tnbuf, obm, obt,
                  sl0, sl1, sn0, sn1, stl):
    wid = lax.axis_index("c") * 16 + lax.axis_index("s")
    g = wid // 4
    q = wid % 4
    row0 = pl.multiple_of(g * GROWS, 8)
    qbase = q * QBASE

    pltpu.sync_copy(a_hbm.at[pl.ds(row0 * 16, GROWS * 16)], abuf)
    pltpu.sync_copy(rs_hbm.at[pl.ds(row0 * 16, GROWS * 16)], rsbuf)
    nrow = [rsbuf[pl.ds(r * 16, 16)][0] for r in range(GROWS)]

    lsems = (sl0, sl1)
    nsems = (sn0, sn1)
    lbufs = (lbuf0, lbuf1)
    nbufs = (nbuf0, nbuf1)

    def issue(c, b):
        col0 = jnp.minimum(qbase + c * CW, VOCAB - TAILW - CW)
        col0 = pl.multiple_of(col0, 128)
        hl = pltpu.async_copy(
            logits_hbm.at[pl.ds(row0, GROWS), pl.ds(col0, CW)],
            lbufs[b], lsems[b])
        hs = [hl]
        for r in range(GROWS):
            noff = pl.multiple_of(nrow[r] * VOCAB, 8) + col0
            hs.append(pltpu.async_copy(ln_hbm.at[pl.ds(noff, CW)],
                                       nbufs[b].at[pl.ds(r * CW, CW)],
                                       nsems[b]))
        return hs

    tail_hs = [pltpu.async_copy(tail_hbm.at[pl.ds(row0 * TAILW, GROWS * TAILW)],
                                tlbuf, stl)]
    for r in range(GROWS):
        noff = pl.multiple_of(nrow[r] * VOCAB, 8) + TAIL0
        tail_hs.append(pltpu.async_copy(ln_hbm.at[pl.ds(noff, TAILW)],
                                        tnbuf.at[pl.ds(r * TAILW, TAILW)], stl))

    lane = lax.iota(jnp.int32, 16)
    neg_inf = jnp.full((16,), -jnp.inf, dtype=jnp.float32)
    zero_i = jnp.zeros((16,), dtype=jnp.int32)
    a_vecs = [abuf[pl.ds(r * 16, 16)] for r in range(GROWS)]

    inflight = [issue(0, 0), issue(1, 1)]

    def scan_chunk(c_idx, b, accs):
        lref = lbufs[b]
        nref = nbufs[b]
        col0 = qbase + c_idx * CW
        new_accs = []
        for r in range(GROWS):
            vm_r, im_r = accs[r]

            def body(i, carry, r=r):
                vms, ims, vidxs = carry
                base = i * (16 * STREAMS)
                nvm, nim, nvi = [], [], []
                for j in range(STREAMS):
                    off = pl.multiple_of(base + j * 16, 16)
                    lv = lref[r, pl.ds(off, 16)]
                    nv = nref[pl.ds(r * CW + off, 16)]
                    score = lv * a_vecs[r] - nv
                    upd = score > vms[j]
                    nvm.append(jnp.where(upd, score, vms[j]))
                    nim.append(jnp.where(upd, vidxs[j], ims[j]))
                    nvi.append(vidxs[j] + 16 * STREAMS)
                return tuple(nvm), tuple(nim), tuple(nvi)

            vidxs = tuple(lane + (col0 + j * 16) for j in range(STREAMS))
            vms0 = (vm_r,) + (neg_inf,) * (STREAMS - 1)
            ims0 = (im_r,) + (zero_i,) * (STREAMS - 1)
            vms, ims, _ = lax.fori_loop(0, GROUPS, body, (vms0, ims0, vidxs))
            vm_r, im_r = vms[0], ims[0]
            for j in range(1, STREAMS):
                vm_r, im_r = _merge(vm_r, im_r, vms[j], ims[j])
            new_accs.append((vm_r, im_r))
        return new_accs

    accs = [(neg_inf, zero_i)] * GROWS
    for k in range(NCHUNK // 2):
        for h in inflight.pop(0):
            h.wait()
        accs = scan_chunk(2 * k, 0, accs)
        inflight.append(issue(2 * k + 2, 0))
        for h in inflight.pop(0):
            h.wait()
        accs = scan_chunk(2 * k + 1, 1, accs)
        inflight.append(issue(2 * k + 3, 1))
    for hs in inflight:
        for h in hs:
            h.wait()

    for h in tail_hs:
        h.wait()
    for r in range(GROWS):
        vm_r, im_r = accs[r]
        for v in range(TAILW // 16):
            lv = tlbuf[pl.ds(r * TAILW + v * 16, 16)]
            nv = tnbuf[pl.ds(r * TAILW + v * 16, 16)]
            score = lv * a_vecs[r] - nv
            vm_r, im_r = _merge(vm_r, im_r, score, lane + (TAIL0 + v * 16))
        accs[r] = (vm_r, im_r)

    mvec = jnp.zeros((16,), dtype=jnp.float32)
    tvec = zero_i
    for r in range(GROWS):
        vm_r, im_r = accs[r]
        m, tok = vm_r[0], im_r[0]
        for k in range(1, 16):
            v, i = vm_r[k], im_r[k]
            take = (v > m) | ((v == m) & (i < tok))
            m = jnp.where(take, v, m)
            tok = jnp.where(take, i, tok)
        mvec = jnp.where(lane == r, m, mvec)
        tvec = jnp.where(lane == r, tok, tvec)
    obm[...] = mvec
    obt[...] = tvec
    off = pl.ds(pl.multiple_of(wid * 16, 8), 16)
    pltpu.sync_copy(obm, outm_hbm.at[off])
    pltpu.sync_copy(obt, outt_hbm.at[off])


@jax.jit
def _sampler(logits, ln, tail, a16, rs16):
    mesh = plsc.VectorSubcoreMesh(core_axis_name="c", subcore_axis_name="s")
    run = functools.partial(
        pl.kernel,
        out_type=(jax.ShapeDtypeStruct((NWORKERS * 16,), jnp.float32),
                  jax.ShapeDtypeStruct((NWORKERS * 16,), jnp.int32)),
        mesh=mesh,
        scratch_types=[
            pltpu.VMEM((GROWS, CW), jnp.float32),
            pltpu.VMEM((GROWS, CW), jnp.float32),
            pltpu.VMEM((GROWS * CW,), jnp.float32),
            pltpu.VMEM((GROWS * CW,), jnp.float32),
            pltpu.VMEM((GROWS * 16,), jnp.float32),
            pltpu.VMEM((GROWS * 16,), jnp.int32),
            pltpu.VMEM((GROWS * TAILW,), jnp.float32),
            pltpu.VMEM((GROWS * TAILW,), jnp.float32),
            pltpu.VMEM((16,), jnp.float32),
            pltpu.VMEM((16,), jnp.int32),
            pltpu.SemaphoreType.DMA,
            pltpu.SemaphoreType.DMA,
            pltpu.SemaphoreType.DMA,
            pltpu.SemaphoreType.DMA,
            pltpu.SemaphoreType.DMA,
        ],
    )(_sampler_body)
    return run(logits, ln, tail, a16, rs16)


def kernel(logits, temperatures):
    ln = jnp.asarray(_LN_NOISE)
    greedy = temperatures < 1e-5
    a = jnp.where(greedy, jnp.float32(1.0), 1.0 / temperatures)
    rs = jnp.where(greedy, jnp.int32(ROWS), jnp.arange(ROWS, dtype=jnp.int32))
    a16 = jnp.broadcast_to(a[:, None], (ROWS, 16)).reshape(-1)
    rs16 = jnp.broadcast_to(rs[:, None], (ROWS, 16)).reshape(-1)
    tail = logits[:, TAIL0:].reshape(-1)
    outm, outt = _sampler(logits, ln, tail, a16, rs16)
    mm = outm.reshape(GROWS, 4, 16)[:, :, :GROWS]
    tt = outt.reshape(GROWS, 4, 16)[:, :, :GROWS]
    m, t = mm[:, 0], tt[:, 0]
    for qq in range(1, 4):
        take = (mm[:, qq] > m) | ((mm[:, qq] == m) & (tt[:, qq] < t))
        m = jnp.where(take, mm[:, qq], m)
        t = jnp.where(take, tt[:, qq], t)
    return t.reshape(ROWS)


# speedup vs baseline: 1.0810x; 1.0810x over previous
import functools

import jax
import jax.numpy as jnp
import numpy as np
from jax import lax
from jax.experimental import pallas as pl
from jax.experimental.pallas import tpu as pltpu
from jax.experimental.pallas import tpu_sc as plsc

ROWS = 64
VOCAB = 100000
QBASE = 24960
QW = 25088
CW = 1792
NCHUNK = QW // CW
TAIL0 = 99968
TAILW = VOCAB - TAIL0
STREAMS = 4
GROUPS = CW // 16 // STREAMS
NWORKERS = 32
GROWS = 8


def _threefry2x32(k0, k1, x0, x1):
    def rotl(x, r):
        return ((x << np.uint32(r)) | (x >> np.uint32(32 - r))).astype(np.uint32)
    rot = ((13, 15, 26, 6), (17, 29, 16, 24))
    ks = [np.uint32(k0), np.uint32(k1),
          np.uint32(np.uint32(k0) ^ np.uint32(k1) ^ np.uint32(0x1BD11BDA))]
    x0 = (x0 + ks[0]).astype(np.uint32)
    x1 = (x1 + ks[1]).astype(np.uint32)
    for i in range(5):
        for r in rot[i % 2]:
            x0 = (x0 + x1).astype(np.uint32)
            x1 = rotl(x1, r)
            x1 = x1 ^ x0
        x0 = (x0 + ks[(i + 1) % 3]).astype(np.uint32)
        x1 = (x1 + ks[(i + 2) % 3] + np.uint32(i + 1)).astype(np.uint32)
    return x0, x1


def _compute_ln_noise() -> np.ndarray:
    size = ROWS * VOCAB
    idx = np.arange(size, dtype=np.uint64)
    x0 = (idx >> np.uint64(32)).astype(np.uint32)
    x1 = (idx & np.uint64(0xFFFFFFFF)).astype(np.uint32)
    r0, r1 = _threefry2x32(np.uint32(0), np.uint32(42), x0, x1)
    bits = r0 ^ r1
    u = ((bits >> np.uint32(9)) | np.uint32(0x3F800000)).view(np.float32) \
        - np.float32(1.0)
    u = np.maximum(u, np.float32(0.0))
    noise = (-np.log1p(-u)).astype(np.float32)
    noise = np.maximum(noise, np.float32(1e-10))
    ln = np.log(noise).astype(np.float32)
    return np.concatenate([ln, np.zeros(VOCAB, np.float32)])


_LN_NOISE = _compute_ln_noise()


def _merge(vm, im, vm2, im2):
    take = (vm2 > vm) | ((vm2 == vm) & (im2 < im))
    return jnp.where(take, vm2, vm), jnp.where(take, im2, im)


def _sampler_body(logits_hbm, ln_hbm, tail_hbm, a_hbm, rs_hbm,
                  outm_hbm, outt_hbm,
                  lbuf0, lbuf1, nbuf0, nbuf1, abuf, rsbuf,
                  tlbuf, tnbuf, obm, obt,
                  sl0, sl1, sn0, sn1, stl):
    wid = lax.axis_index("c") * 16 + lax.axis_index("s")
    g = wid // 4
    q = wid % 4
    row0 = pl.multiple_of(g * GROWS, 8)
    qbase = q * QBASE

    pltpu.sync_copy(a_hbm.at[pl.ds(row0 * 16, GROWS * 16)], abuf)
    pltpu.sync_copy(rs_hbm.at[pl.ds(row0 * 16, GROWS * 16)], rsbuf)
    nrow = [rsbuf[pl.ds(r * 16, 16)][0] for r in range(GROWS)]

    lsems = (sl0, sl1)
    nsems = (sn0, sn1)
    lbufs = (lbuf0, lbuf1)
    nbufs = (nbuf0, nbuf1)

    def issue(c, b):
        col0 = jnp.minimum(qbase + c * CW, VOCAB - TAILW - CW)
        col0 = pl.multiple_of(col0, 128)
        hl = pltpu.async_copy(
            logits_hbm.at[pl.ds(row0, GROWS), pl.ds(col0, CW)],
            lbufs[b], lsems[b])
        hs = [hl]
        for r in range(GROWS):
            noff = pl.multiple_of(nrow[r] * VOCAB, 8) + col0
            hs.append(pltpu.async_copy(ln_hbm.at[pl.ds(noff, CW)],
                                       nbufs[b].at[pl.ds(r * CW, CW)],
                                       nsems[b]))
        return hs

    tail_hs = [pltpu.async_copy(tail_hbm.at[pl.ds(row0 * TAILW, GROWS * TAILW)],
                                tlbuf, stl)]
    for r in range(GROWS):
        noff = pl.multiple_of(nrow[r] * VOCAB, 8) + TAIL0
        tail_hs.append(pltpu.async_copy(ln_hbm.at[pl.ds(noff, TAILW)],
                                        tnbuf.at[pl.ds(r * TAILW, TAILW)], stl))

    lane = lax.iota(jnp.int32, 16)
    neg_inf = jnp.full((16,), -jnp.inf, dtype=jnp.float32)
    zero_i = jnp.zeros((16,), dtype=jnp.int32)
    a_vecs = [abuf[pl.ds(r * 16, 16)] for r in range(GROWS)]

    issue(0, 0)
    issue(1, 1)

    def wait_slot(b):
        pltpu.make_async_copy(
            logits_hbm.at[pl.ds(row0, GROWS), pl.ds(0, CW)],
            lbufs[b], lsems[b]).wait()
        for r in range(GROWS):
            pltpu.make_async_copy(ln_hbm.at[pl.ds(0, CW)],
                                  nbufs[b].at[pl.ds(r * CW, CW)],
                                  nsems[b]).wait()

    def scan_chunk(c_idx, b, accs):
        lref = lbufs[b]
        nref = nbufs[b]
        col0 = qbase + c_idx * CW
        new_accs = []
        for r in range(GROWS):
            vm_r, im_r = accs[r]

            def body(i, carry, r=r):
                vms, ims, vidxs = carry
                base = i * (16 * STREAMS)
                nvm, nim, nvi = [], [], []
                for j in range(STREAMS):
                    off = pl.multiple_of(base + j * 16, 16)
                    lv = lref[r, pl.ds(off, 16)]
                    nv = nref[pl.ds(r * CW + off, 16)]
                    score = lv * a_vecs[r] - nv
                    upd = score > vms[j]
                    nvm.append(jnp.where(upd, score, vms[j]))
                    nim.append(jnp.where(upd, vidxs[j], ims[j]))
                    nvi.append(vidxs[j] + 16 * STREAMS)
                return tuple(nvm), tuple(nim), tuple(nvi)

            vidxs = tuple(lane + (col0 + j * 16) for j in range(STREAMS))
            vms0 = (vm_r,) + (neg_inf,) * (STREAMS - 1)
            ims0 = (im_r,) + (zero_i,) * (STREAMS - 1)
            vms, ims, _ = lax.fori_loop(0, GROUPS, body, (vms0, ims0, vidxs),
                                        unroll=2)
            vm_r, im_r = vms[0], ims[0]
            for j in range(1, STREAMS):
                vm_r, im_r = _merge(vm_r, im_r, vms[j], ims[j])
            new_accs.append((vm_r, im_r))
        return new_accs

    def chunk_pair(k, carry):
        accs = [(carry[2 * r], carry[2 * r + 1]) for r in range(GROWS)]
        wait_slot(0)
        accs = scan_chunk(2 * k, 0, accs)
        issue(2 * k + 2, 0)
        wait_slot(1)
        accs = scan_chunk(2 * k + 1, 1, accs)
        issue(2 * k + 3, 1)
        return tuple(x for acc in accs for x in acc)

    carry0 = tuple(x for _ in range(GROWS) for x in (neg_inf, zero_i))
    carry = lax.fori_loop(0, NCHUNK // 2, chunk_pair, carry0)
    accs = [(carry[2 * r], carry[2 * r + 1]) for r in range(GROWS)]
    wait_slot(0)
    wait_slot(1)

    for h in tail_hs:
        h.wait()
    for r in range(GROWS):
        vm_r, im_r = accs[r]
        for v in range(TAILW // 16):
            lv = tlbuf[pl.ds(r * TAILW + v * 16, 16)]
            nv = tnbuf[pl.ds(r * TAILW + v * 16, 16)]
            score = lv * a_vecs[r] - nv
            vm_r, im_r = _merge(vm_r, im_r, score, lane + (TAIL0 + v * 16))
        accs[r] = (vm_r, im_r)

    mvec = jnp.zeros((16,), dtype=jnp.float32)
    tvec = zero_i
    for r in range(GROWS):
        vm_r, im_r = accs[r]
        m, tok = vm_r[0], im_r[0]
        for k in range(1, 16):
            v, i = vm_r[k], im_r[k]
            take = (v > m) | ((v == m) & (i < tok))
            m = jnp.where(take, v, m)
            tok = jnp.where(take, i, tok)
        mvec = jnp.where(lane == r, m, mvec)
        tvec = jnp.where(lane == r, tok, tvec)
    obm[...] = mvec
    obt[...] = tvec
    off = pl.ds(pl.multiple_of(wid * 16, 8), 16)
    pltpu.sync_copy(obm, outm_hbm.at[off])
    pltpu.sync_copy(obt, outt_hbm.at[off])


@jax.jit
def _sampler(logits, ln, tail, a16, rs16):
    mesh = plsc.VectorSubcoreMesh(core_axis_name="c", subcore_axis_name="s")
    run = functools.partial(
        pl.kernel,
        out_type=(jax.ShapeDtypeStruct((NWORKERS * 16,), jnp.float32),
                  jax.ShapeDtypeStruct((NWORKERS * 16,), jnp.int32)),
        mesh=mesh,
        scratch_types=[
            pltpu.VMEM((GROWS, CW), jnp.float32),
            pltpu.VMEM((GROWS, CW), jnp.float32),
            pltpu.VMEM((GROWS * CW,), jnp.float32),
            pltpu.VMEM((GROWS * CW,), jnp.float32),
            pltpu.VMEM((GROWS * 16,), jnp.float32),
            pltpu.VMEM((GROWS * 16,), jnp.int32),
            pltpu.VMEM((GROWS * TAILW,), jnp.float32),
            pltpu.VMEM((GROWS * TAILW,), jnp.float32),
            pltpu.VMEM((16,), jnp.float32),
            pltpu.VMEM((16,), jnp.int32),
            pltpu.SemaphoreType.DMA,
            pltpu.SemaphoreType.DMA,
            pltpu.SemaphoreType.DMA,
            pltpu.SemaphoreType.DMA,
            pltpu.SemaphoreType.DMA,
        ],
    )(_sampler_body)
    return run(logits, ln, tail, a16, rs16)


def kernel(logits, temperatures):
    ln = jnp.asarray(_LN_NOISE)
    greedy = temperatures < 1e-5
    a = jnp.where(greedy, jnp.float32(1.0), 1.0 / temperatures)
    rs = jnp.where(greedy, jnp.int32(ROWS), jnp.arange(ROWS, dtype=jnp.int32))
    a16 = jnp.broadcast_to(a[:, None], (ROWS, 16)).reshape(-1)
    rs16 = jnp.broadcast_to(rs[:, None], (ROWS, 16)).reshape(-1)
    tail = logits[:, TAIL0:].reshape(-1)
    outm, outt = _sampler(logits, ln, tail, a16, rs16)
    mm = outm.reshape(GROWS, 4, 16)[:, :, :GROWS]
    tt = outt.reshape(GROWS, 4, 16)[:, :, :GROWS]
    m, t = mm[:, 0], tt[:, 0]
    for qq in range(1, 4):
        take = (mm[:, qq] > m) | ((mm[:, qq] == m) & (tt[:, qq] < t))
        m = jnp.where(take, mm[:, qq], m)
        t = jnp.where(take, tt[:, qq], t)
    return t.reshape(ROWS)


# speedup vs baseline: 1.0885x; 1.0069x over previous
import functools

import jax
import jax.numpy as jnp
import numpy as np
from jax import lax
from jax.experimental import pallas as pl
from jax.experimental.pallas import tpu as pltpu
from jax.experimental.pallas import tpu_sc as plsc

ROWS = 64
VOCAB = 100000
QBASE = 24960
QW = 25088
CW = 1792
NCHUNK = QW // CW
TAIL0 = 99968
TAILW = VOCAB - TAIL0
STREAMS = 4
GROUPS = CW // 16 // STREAMS
NWORKERS = 32
GROWS = 8


def _threefry2x32(k0, k1, x0, x1):
    def rotl(x, r):
        return ((x << np.uint32(r)) | (x >> np.uint32(32 - r))).astype(np.uint32)
    rot = ((13, 15, 26, 6), (17, 29, 16, 24))
    ks = [np.uint32(k0), np.uint32(k1),
          np.uint32(np.uint32(k0) ^ np.uint32(k1) ^ np.uint32(0x1BD11BDA))]
    x0 = (x0 + ks[0]).astype(np.uint32)
    x1 = (x1 + ks[1]).astype(np.uint32)
    for i in range(5):
        for r in rot[i % 2]:
            x0 = (x0 + x1).astype(np.uint32)
            x1 = rotl(x1, r)
            x1 = x1 ^ x0
        x0 = (x0 + ks[(i + 1) % 3]).astype(np.uint32)
        x1 = (x1 + ks[(i + 2) % 3] + np.uint32(i + 1)).astype(np.uint32)
    return x0, x1


def _compute_ln_noise() -> np.ndarray:
    size = ROWS * VOCAB
    idx = np.arange(size, dtype=np.uint64)
    x0 = (idx >> np.uint64(32)).astype(np.uint32)
    x1 = (idx & np.uint64(0xFFFFFFFF)).astype(np.uint32)
    r0, r1 = _threefry2x32(np.uint32(0), np.uint32(42), x0, x1)
    bits = r0 ^ r1
    u = ((bits >> np.uint32(9)) | np.uint32(0x3F800000)).view(np.float32) \
        - np.float32(1.0)
    u = np.maximum(u, np.float32(0.0))
    noise = (-np.log1p(-u)).astype(np.float32)
    noise = np.maximum(noise, np.float32(1e-10))
    ln = np.log(noise).astype(np.float32)
    return np.concatenate([ln, np.zeros(VOCAB, np.float32)])


_LN_NOISE = _compute_ln_noise()


def _merge(vm, im, vm2, im2):
    take = (vm2 > vm) | ((vm2 == vm) & (im2 < im))
    return jnp.where(take, vm2, vm), jnp.where(take, im2, im)


def _sampler_body(logits_hbm, ln_hbm, tail_hbm, a_hbm, rs_hbm,
                  outm_hbm, outt_hbm,
                  lbuf0, lbuf1, nbuf0, nbuf1, abuf, rsbuf,
                  tlbuf, tnbuf, obm, obt,
                  sl0, sl1, sn0, sn1, stl):
    wid = lax.axis_index("c") * 16 + lax.axis_index("s")
    g = wid // 4
    q = wid % 4
    row0 = pl.multiple_of(g * GROWS, 8)
    qbase = q * QBASE

    lsems = (sl0, sl1)
    nsems = (sn0, sn1)
    lbufs = (lbuf0, lbuf1)
    nbufs = (nbuf0, nbuf1)

    def chunk_col(c):
        col0 = jnp.minimum(qbase + c * CW, VOCAB - TAILW - CW)
        return pl.multiple_of(col0, 128)

    def issue_l(c, b):
        pltpu.async_copy(
            logits_hbm.at[pl.ds(row0, GROWS), pl.ds(chunk_col(c), CW)],
            lbufs[b], lsems[b])

    issue_l(0, 0)
    issue_l(1, 1)

    pltpu.sync_copy(a_hbm.at[pl.ds(row0 * 16, GROWS * 16)], abuf)
    pltpu.sync_copy(rs_hbm.at[pl.ds(row0 * 16, GROWS * 16)], rsbuf)
    nrow = [rsbuf[pl.ds(r * 16, 16)][0] for r in range(GROWS)]
    nbase = [pl.multiple_of(nrow[r] * VOCAB, 8) for r in range(GROWS)]

    def issue_n(c, b):
        col0 = chunk_col(c)
        for r in range(GROWS):
            pltpu.async_copy(ln_hbm.at[pl.ds(nbase[r] + col0, CW)],
                             nbufs[b].at[pl.ds(r * CW, CW)], nsems[b])

    def issue(c, b):
        issue_l(c, b)
        issue_n(c, b)

    issue_n(0, 0)
    issue_n(1, 1)

    tail_hs = [pltpu.async_copy(tail_hbm.at[pl.ds(row0 * TAILW, GROWS * TAILW)],
                                tlbuf, stl)]
    for r in range(GROWS):
        tail_hs.append(pltpu.async_copy(ln_hbm.at[pl.ds(nbase[r] + TAIL0, TAILW)],
                                        tnbuf.at[pl.ds(r * TAILW, TAILW)], stl))

    lane = lax.iota(jnp.int32, 16)
    neg_inf = jnp.full((16,), -jnp.inf, dtype=jnp.float32)
    zero_i = jnp.zeros((16,), dtype=jnp.int32)
    a_vecs = [abuf[pl.ds(r * 16, 16)] for r in range(GROWS)]

    def wait_slot(b):
        pltpu.make_async_copy(
            logits_hbm.at[pl.ds(row0, GROWS), pl.ds(0, CW)],
            lbufs[b], lsems[b]).wait()
        for r in range(GROWS):
            pltpu.make_async_copy(ln_hbm.at[pl.ds(0, CW)],
                                  nbufs[b].at[pl.ds(r * CW, CW)],
                                  nsems[b]).wait()

    def scan_chunk(c_idx, b, accs):
        lref = lbufs[b]
        nref = nbufs[b]
        col0 = qbase + c_idx * CW
        new_accs = []
        for r in range(GROWS):
            vm_r, im_r = accs[r]

            def body(i, carry, r=r):
                vms, ims, vidxs = carry
                base = i * (16 * STREAMS)
                nvm, nim, nvi = [], [], []
                for j in range(STREAMS):
                    off = pl.multiple_of(base + j * 16, 16)
                    lv = lref[r, pl.ds(off, 16)]
                    nv = nref[pl.ds(r * CW + off, 16)]
                    score = lv * a_vecs[r] - nv
                    upd = score > vms[j]
                    nvm.append(jnp.where(upd, score, vms[j]))
                    nim.append(jnp.where(upd, vidxs[j], ims[j]))
                    nvi.append(vidxs[j] + 16 * STREAMS)
                return tuple(nvm), tuple(nim), tuple(nvi)

            vidxs = tuple(lane + (col0 + j * 16) for j in range(STREAMS))
            vms0 = (vm_r,) + (neg_inf,) * (STREAMS - 1)
            ims0 = (im_r,) + (zero_i,) * (STREAMS - 1)
            vms, ims, _ = lax.fori_loop(0, GROUPS, body, (vms0, ims0, vidxs),
                                        unroll=2)
            vm_r, im_r = vms[0], ims[0]
            for j in range(1, STREAMS):
                vm_r, im_r = _merge(vm_r, im_r, vms[j], ims[j])
            new_accs.append((vm_r, im_r))
        return new_accs

    def chunk_pair(k, carry):
        accs = [(carry[2 * r], carry[2 * r + 1]) for r in range(GROWS)]
        wait_slot(0)
        accs = scan_chunk(2 * k, 0, accs)
        issue(2 * k + 2, 0)
        wait_slot(1)
        accs = scan_chunk(2 * k + 1, 1, accs)
        issue(2 * k + 3, 1)
        return tuple(x for acc in accs for x in acc)

    carry0 = tuple(x for _ in range(GROWS) for x in (neg_inf, zero_i))
    carry = lax.fori_loop(0, NCHUNK // 2, chunk_pair, carry0)
    accs = [(carry[2 * r], carry[2 * r + 1]) for r in range(GROWS)]
    wait_slot(0)
    wait_slot(1)

    for h in tail_hs:
        h.wait()
    for r in range(GROWS):
        vm_r, im_r = accs[r]
        for v in range(TAILW // 16):
            lv = tlbuf[pl.ds(r * TAILW + v * 16, 16)]
            nv = tnbuf[pl.ds(r * TAILW + v * 16, 16)]
            score = lv * a_vecs[r] - nv
            vm_r, im_r = _merge(vm_r, im_r, score, lane + (TAIL0 + v * 16))
        accs[r] = (vm_r, im_r)

    mvec = jnp.zeros((16,), dtype=jnp.float32)
    tvec = zero_i
    for r in range(GROWS):
        vm_r, im_r = accs[r]
        m, tok = vm_r[0], im_r[0]
        for k in range(1, 16):
            v, i = vm_r[k], im_r[k]
            take = (v > m) | ((v == m) & (i < tok))
            m = jnp.where(take, v, m)
            tok = jnp.where(take, i, tok)
        mvec = jnp.where(lane == r, m, mvec)
        tvec = jnp.where(lane == r, tok, tvec)
    obm[...] = mvec
    obt[...] = tvec
    off = pl.ds(pl.multiple_of(wid * 16, 8), 16)
    pltpu.sync_copy(obm, outm_hbm.at[off])
    pltpu.sync_copy(obt, outt_hbm.at[off])


@jax.jit
def _sampler(logits, ln, tail, a16, rs16):
    mesh = plsc.VectorSubcoreMesh(core_axis_name="c", subcore_axis_name="s")
    run = functools.partial(
        pl.kernel,
        out_type=(jax.ShapeDtypeStruct((NWORKERS * 16,), jnp.float32),
                  jax.ShapeDtypeStruct((NWORKERS * 16,), jnp.int32)),
        mesh=mesh,
        scratch_types=[
            pltpu.VMEM((GROWS, CW), jnp.float32),
            pltpu.VMEM((GROWS, CW), jnp.float32),
            pltpu.VMEM((GROWS * CW,), jnp.float32),
            pltpu.VMEM((GROWS * CW,), jnp.float32),
            pltpu.VMEM((GROWS * 16,), jnp.float32),
            pltpu.VMEM((GROWS * 16,), jnp.int32),
            pltpu.VMEM((GROWS * TAILW,), jnp.float32),
            pltpu.VMEM((GROWS * TAILW,), jnp.float32),
            pltpu.VMEM((16,), jnp.float32),
            pltpu.VMEM((16,), jnp.int32),
            pltpu.SemaphoreType.DMA,
            pltpu.SemaphoreType.DMA,
            pltpu.SemaphoreType.DMA,
            pltpu.SemaphoreType.DMA,
            pltpu.SemaphoreType.DMA,
        ],
    )(_sampler_body)
    return run(logits, ln, tail, a16, rs16)


def kernel(logits, temperatures):
    ln = jnp.asarray(_LN_NOISE)
    greedy = temperatures < 1e-5
    a = jnp.where(greedy, jnp.float32(1.0), 1.0 / temperatures)
    rs = jnp.where(greedy, jnp.int32(ROWS), jnp.arange(ROWS, dtype=jnp.int32))
    a16 = jnp.broadcast_to(a[:, None], (ROWS, 16)).reshape(-1)
    rs16 = jnp.broadcast_to(rs[:, None], (ROWS, 16)).reshape(-1)
    tail = logits[:, TAIL0:].reshape(-1)
    outm, outt = _sampler(logits, ln, tail, a16, rs16)
    mm = outm.reshape(GROWS, 4, 16)[:, :, :GROWS]
    tt = outt.reshape(GROWS, 4, 16)[:, :, :GROWS]
    m, t = mm[:, 0], tt[:, 0]
    for qq in range(1, 4):
        take = (mm[:, qq] > m) | ((mm[:, qq] == m) & (tt[:, qq] < t))
        m = jnp.where(take, mm[:, qq], m)
        t = jnp.where(take, tt[:, qq], t)
    return t.reshape(ROWS)
